# baseline (speedup 1.0000x reference)
import functools

import jax
import jax.numpy as jnp
from jax import lax
from jax.experimental import pallas as pl
from jax.experimental.pallas import tpu as pltpu
from jax.experimental.pallas import tpu_sc as plsc

VOCAB = 1000000
EMB = 32
HID = 128
NCLS = 100
B = 16384
L = 200

_NC = 2
_NS = 16
_NW = _NC * _NS
_BPW = B // _NW
_C = 8
_NCHUNK = _BPW // _C
_IDX_PER_CHUNK = _C * L

_RW = 1920
_RB = _RW // 4


def _tc_reformat(embedT):
    grid = (-(-VOCAB // _RW),)

    def body(in_ref, o_ref):
        t = in_ref[...].T.reshape(_RB, 4, EMB)
        o_ref[...] = jnp.concatenate([t[:, j, :] for j in range(4)], axis=1)

    return pl.pallas_call(
        body,
        grid=grid,
        in_specs=[pl.BlockSpec((EMB, _RW), lambda i: (0, i))],
        out_specs=pl.BlockSpec((_RB, 128), lambda i: (i, 0)),
        out_shape=jax.ShapeDtypeStruct((VOCAB // 4, 128), jnp.float32),
    )(embedT)


def _sc_segment_sum(x, embed_lin):
    mesh = plsc.VectorSubcoreMesh(core_axis_name="c", subcore_axis_name="s")

    @functools.partial(
        pl.kernel,
        out_type=jax.ShapeDtypeStruct((B, EMB), jnp.float32),
        mesh=mesh,
        compiler_params=pltpu.CompilerParams(use_tc_tiling_on_sc=False),
        scratch_types=[
            pltpu.VMEM((_C, L), jnp.int32),
            pltpu.VMEM((_C, L), jnp.int32),
            pltpu.VMEM((_IDX_PER_CHUNK, EMB), jnp.float32),
            pltpu.VMEM((_IDX_PER_CHUNK, EMB), jnp.float32),
            pltpu.VMEM((_C, EMB), jnp.float32),
            pltpu.SemaphoreType.DMA,
            pltpu.SemaphoreType.DMA,
        ],
    )
    def k(x_hbm, embed_hbm, out_hbm, idx0, idx1, rows0, rows1, sum_v,
          sem0, sem1):
        wid = lax.axis_index("s") * _NC + lax.axis_index("c")
        worker_row0 = wid * _BPW

        def fire(idx_v, rows_v, sem, g):
            pltpu.sync_copy(x_hbm.at[pl.ds(worker_row0 + g * _C, _C)], idx_v)
            for c in range(_C):
                pltpu.async_copy(
                    embed_hbm.at[idx_v.at[c, pl.ds(0, 128)]],
                    rows_v.at[pl.ds(c * L, 128)], sem)
                pltpu.async_copy(
                    embed_hbm.at[idx_v.at[c, pl.ds(128, L - 128)]],
                    rows_v.at[pl.ds(c * L + 128, L - 128)], sem)

        def drain(rows_v, sem):
            pltpu.make_async_copy(
                embed_hbm.at[pl.ds(0, _IDX_PER_CHUNK)], rows_v, sem).wait()

        def compute_store(rows_v, g):
            for c in range(_C):
                zero = jnp.zeros((16,), jnp.float32)
                accs = (zero,) * 8

                def row_body(j, a, c=c, rows_v=rows_v):
                    base = c * L + j * 4
                    lo = [a[u] + rows_v[base + u, pl.ds(0, 16)]
                          for u in range(4)]
                    hi = [a[4 + u] + rows_v[base + u, pl.ds(16, 16)]
                          for u in range(4)]
                    return tuple(lo + hi)

                accs = lax.fori_loop(0, L // 4, row_body, accs)
                sum_v[c, pl.ds(0, 16)] = (accs[0] + accs[1]) + (accs[2] + accs[3])
                sum_v[c, pl.ds(16, 16)] = (accs[4] + accs[5]) + (accs[6] + accs[7])
            pltpu.sync_copy(sum_v, out_hbm.at[pl.ds(worker_row0 + g * _C, _C)])

        fire(idx0, rows0, sem0, 0)

        def body(i, carry):
            g0 = 2 * i
            g1 = g0 + 1
            fire(idx1, rows1, sem1, g1)
            drain(rows0, sem0)
            compute_store(rows0, g0)

            @pl.when(g0 + 2 < _NCHUNK)
            def _():
                fire(idx0, rows0, sem0, g0 + 2)

            drain(rows1, sem1)
            compute_store(rows1, g1)
            return carry

        lax.fori_loop(0, _NCHUNK // 2, body, 0)

    return k(x, embed_lin)


def _tc_mlp(xT, summed, W1T, b1, W2T, b2):
    blk = 2048
    grid = (B // blk,)

    def body(xt_ref, s_ref, w1t_ref, b1_ref, w2t_ref, b2_ref, o_ref):
        cnt = jnp.sum((xt_ref[...] != 0).astype(jnp.float32), axis=0,
                      keepdims=True)
        avgT = s_ref[...].T / jnp.maximum(cnt, 1e-6)
        h = jnp.dot(w1t_ref[...], avgT, preferred_element_type=jnp.float32)
        h = jnp.maximum(h + b1_ref[...], 0.0)
        o = jnp.dot(w2t_ref[...], h, preferred_element_type=jnp.float32)
        o_ref[...] = o + b2_ref[...]

    return pl.pallas_call(
        body,
        grid=grid,
        in_specs=[
            pl.BlockSpec((L, blk), lambda i: (0, i)),
            pl.BlockSpec((blk, EMB), lambda i: (i, 0)),
            pl.BlockSpec((HID, EMB), lambda i: (0, 0)),
            pl.BlockSpec((HID, 1), lambda i: (0, 0)),
            pl.BlockSpec((NCLS, HID), lambda i: (0, 0)),
            pl.BlockSpec((NCLS, 1), lambda i: (0, 0)),
        ],
        out_specs=pl.BlockSpec((NCLS, blk), lambda i: (0, i)),
        out_shape=jax.ShapeDtypeStruct((NCLS, B), jnp.float32),
    )(xT, summed, W1T, b1.reshape(HID, 1), W2T, b2.reshape(NCLS, 1))


def kernel(x, embed, W1, b1, W2, b2):
    x = x.astype(jnp.int32)
    embed_lin = _tc_reformat(embed.T).reshape(VOCAB, EMB)
    summed = _sc_segment_sum(x, embed_lin)
    outT = _tc_mlp(x.T, summed, W1.T, b1, W2.T, b2)
    return outT.T


# speedup vs baseline: 1.5849x; 1.5849x over previous
import functools

import jax
import jax.numpy as jnp
from jax import lax
from jax.experimental import pallas as pl
from jax.experimental.pallas import tpu as pltpu
from jax.experimental.pallas import tpu_sc as plsc

VOCAB = 1000000
EMB = 32
HID = 128
NCLS = 100
B = 16384
L = 200

_NC = 2
_NS = 16
_NW = _NC * _NS
_BPW = B // _NW
_C = 8
_NCHUNK = _BPW // _C
_IDX_PER_CHUNK = _C * L

_RSUP = 512
_RK = 8
_RGRID = -(-VOCAB // _RSUP)
_VPAD = _RGRID * _RSUP


def _tc_reformat(embedT):
    def body(in_ref, o_ref):
        for k in range(_RK):
            t = jnp.concatenate(
                [in_ref[:, _RSUP * k + 128 * j:_RSUP * k + 128 * (j + 1)]
                 for j in range(4)], axis=0)
            o_ref[128 * k:128 * (k + 1), :] = t.T

    return pl.pallas_call(
        body,
        grid=(-(-_RGRID // _RK),),
        in_specs=[pl.BlockSpec((EMB, _RSUP * _RK), lambda i: (0, i))],
        out_specs=pl.BlockSpec((128 * _RK, 128), lambda i: (i, 0)),
        out_shape=jax.ShapeDtypeStruct((_VPAD // 4, 128), jnp.float32),
    )(embedT)


def _tc_xform_idx(x):
    blk = 2048
    grid = (B // blk,)

    def body(x_ref, o_ref):
        v = x_ref[...]
        o_ref[...] = (v & ~511) | ((v << 2) & 511) | ((v >> 7) & 3)

    return pl.pallas_call(
        body,
        grid=grid,
        in_specs=[pl.BlockSpec((blk, L), lambda i: (i, 0))],
        out_specs=pl.BlockSpec((blk, L), lambda i: (i, 0)),
        out_shape=jax.ShapeDtypeStruct((B, L), jnp.int32),
    )(x)


def _sc_segment_sum(x, embed_lin):
    mesh = plsc.VectorSubcoreMesh(core_axis_name="c", subcore_axis_name="s")

    @functools.partial(
        pl.kernel,
        out_type=jax.ShapeDtypeStruct((B, EMB), jnp.float32),
        mesh=mesh,
        compiler_params=pltpu.CompilerParams(use_tc_tiling_on_sc=False),
        scratch_types=[
            pltpu.VMEM((_C, L), jnp.int32),
            pltpu.VMEM((_C, L), jnp.int32),
            pltpu.VMEM((_IDX_PER_CHUNK, EMB), jnp.float32),
            pltpu.VMEM((_IDX_PER_CHUNK, EMB), jnp.float32),
            pltpu.VMEM((_C, EMB), jnp.float32),
            pltpu.SemaphoreType.DMA,
            pltpu.SemaphoreType.DMA,
        ],
    )
    def k(x_hbm, embed_hbm, out_hbm, idx0, idx1, rows0, rows1, sum_v,
          sem0, sem1):
        wid = lax.axis_index("s") * _NC + lax.axis_index("c")
        worker_row0 = wid * _BPW

        def fire(idx_v, rows_v, sem, g):
            pltpu.sync_copy(x_hbm.at[pl.ds(worker_row0 + g * _C, _C)], idx_v)
            for c in range(_C):
                pltpu.async_copy(
                    embed_hbm.at[idx_v.at[c, pl.ds(0, 128)]],
                    rows_v.at[pl.ds(c * L, 128)], sem)
                pltpu.async_copy(
                    embed_hbm.at[idx_v.at[c, pl.ds(128, L - 128)]],
                    rows_v.at[pl.ds(c * L + 128, L - 128)], sem)

        def drain(rows_v, sem):
            pltpu.make_async_copy(
                embed_hbm.at[pl.ds(0, _IDX_PER_CHUNK)], rows_v, sem).wait()

        def compute_store(rows_v, g):
            for c in range(_C):
                zero = jnp.zeros((16,), jnp.float32)
                accs = (zero,) * 8

                def row_body(j, a, c=c, rows_v=rows_v):
                    base = c * L + j * 4
                    lo = [a[u] + rows_v[base + u, pl.ds(0, 16)]
                          for u in range(4)]
                    hi = [a[4 + u] + rows_v[base + u, pl.ds(16, 16)]
                          for u in range(4)]
                    return tuple(lo + hi)

                accs = lax.fori_loop(0, L // 4, row_body, accs)
                sum_v[c, pl.ds(0, 16)] = (accs[0] + accs[1]) + (accs[2] + accs[3])
                sum_v[c, pl.ds(16, 16)] = (accs[4] + accs[5]) + (accs[6] + accs[7])
            pltpu.sync_copy(sum_v, out_hbm.at[pl.ds(worker_row0 + g * _C, _C)])

        fire(idx0, rows0, sem0, 0)

        def body(i, carry):
            g0 = 2 * i
            g1 = g0 + 1
            fire(idx1, rows1, sem1, g1)
            drain(rows0, sem0)
            compute_store(rows0, g0)

            @pl.when(g0 + 2 < _NCHUNK)
            def _():
                fire(idx0, rows0, sem0, g0 + 2)

            drain(rows1, sem1)
            compute_store(rows1, g1)
            return carry

        lax.fori_loop(0, _NCHUNK // 2, body, 0)

    return k(x, embed_lin)


def _tc_mlp(xT, summed, W1T, b1, W2T, b2):
    blk = 2048
    grid = (B // blk,)

    def body(xt_ref, s_ref, w1t_ref, b1_ref, w2t_ref, b2_ref, o_ref):
        cnt = jnp.sum((xt_ref[...] != 0).astype(jnp.float32), axis=0,
                      keepdims=True)
        avgT = s_ref[...].T / jnp.maximum(cnt, 1e-6)
        h = jnp.dot(w1t_ref[...], avgT, preferred_element_type=jnp.float32)
        h = jnp.maximum(h + b1_ref[...], 0.0)
        o = jnp.dot(w2t_ref[...], h, preferred_element_type=jnp.float32)
        o_ref[...] = o + b2_ref[...]

    return pl.pallas_call(
        body,
        grid=grid,
        in_specs=[
            pl.BlockSpec((L, blk), lambda i: (0, i)),
            pl.BlockSpec((blk, EMB), lambda i: (i, 0)),
            pl.BlockSpec((HID, EMB), lambda i: (0, 0)),
            pl.BlockSpec((HID, 1), lambda i: (0, 0)),
            pl.BlockSpec((NCLS, HID), lambda i: (0, 0)),
            pl.BlockSpec((NCLS, 1), lambda i: (0, 0)),
        ],
        out_specs=pl.BlockSpec((NCLS, blk), lambda i: (0, i)),
        out_shape=jax.ShapeDtypeStruct((NCLS, B), jnp.float32),
    )(xT, summed, W1T, b1.reshape(HID, 1), W2T, b2.reshape(NCLS, 1))


def kernel(x, embed, W1, b1, W2, b2):
    x = x.astype(jnp.int32)
    embed_lin = _tc_reformat(embed.T).reshape(_VPAD, EMB)
    fx = _tc_xform_idx(x)
    summed = _sc_segment_sum(fx, embed_lin)
    outT = _tc_mlp(x.T, summed, W1.T, b1, W2.T, b2)
    return outT.T


# speedup vs baseline: 1.8121x; 1.1433x over previous
import functools

import jax
import jax.numpy as jnp
from jax import lax
from jax.experimental import pallas as pl
from jax.experimental.pallas import tpu as pltpu
from jax.experimental.pallas import tpu_sc as plsc

VOCAB = 1000000
EMB = 32
HID = 128
NCLS = 100
B = 16384
L = 200

_NC = 2
_NS = 16
_NW = _NC * _NS
_BPW = B // _NW
_C = 8
_NCHUNK = _BPW // _C
_IDX_PER_CHUNK = _C * L

_RSUP = 512
_RK = 16
_RGRID = -(-VOCAB // _RSUP)
_VPAD = _RGRID * _RSUP


def _tc_reformat(embedT):
    def body(in_ref, o_ref):
        for k in range(_RK):
            t = jnp.concatenate(
                [in_ref[:, _RSUP * k + 128 * j:_RSUP * k + 128 * (j + 1)]
                 for j in range(4)], axis=0)
            o_ref[128 * k:128 * (k + 1), :] = t.T

    return pl.pallas_call(
        body,
        grid=(-(-_RGRID // _RK),),
        in_specs=[pl.BlockSpec((EMB, _RSUP * _RK), lambda i: (0, i))],
        out_specs=pl.BlockSpec((128 * _RK, 128), lambda i: (i, 0)),
        out_shape=jax.ShapeDtypeStruct((_VPAD // 4, 128), jnp.float32),
    )(embedT)


def _tc_xform_idx(x):
    blk = 2048
    grid = (B // blk,)

    def body(x_ref, o_ref):
        v = x_ref[...]
        o_ref[...] = (v & ~511) | ((v << 2) & 511) | ((v >> 7) & 3)

    return pl.pallas_call(
        body,
        grid=grid,
        in_specs=[pl.BlockSpec((blk, L), lambda i: (i, 0))],
        out_specs=pl.BlockSpec((blk, L), lambda i: (i, 0)),
        out_shape=jax.ShapeDtypeStruct((B, L), jnp.int32),
    )(x)


def _sc_segment_sum(x, embed_lin):
    mesh = plsc.VectorSubcoreMesh(core_axis_name="c", subcore_axis_name="s")

    @functools.partial(
        pl.kernel,
        out_type=jax.ShapeDtypeStruct((B, EMB), jnp.float32),
        mesh=mesh,
        compiler_params=pltpu.CompilerParams(use_tc_tiling_on_sc=False),
        scratch_types=[
            pltpu.VMEM((_C, L), jnp.int32),
            pltpu.VMEM((_C, L), jnp.int32),
            pltpu.VMEM((_IDX_PER_CHUNK, EMB), jnp.float32),
            pltpu.VMEM((_IDX_PER_CHUNK, EMB), jnp.float32),
            pltpu.VMEM((_C, EMB), jnp.float32),
            pltpu.SemaphoreType.DMA,
            pltpu.SemaphoreType.DMA,
        ],
    )
    def k(x_hbm, embed_hbm, out_hbm, idx0, idx1, rows0, rows1, sum_v,
          sem0, sem1):
        wid = lax.axis_index("s") * _NC + lax.axis_index("c")
        worker_row0 = wid * _BPW

        def fire(idx_v, rows_v, sem, g):
            pltpu.sync_copy(x_hbm.at[pl.ds(worker_row0 + g * _C, _C)], idx_v)
            for c in range(_C):
                pltpu.async_copy(
                    embed_hbm.at[idx_v.at[c, pl.ds(0, 128)]],
                    rows_v.at[pl.ds(c * L, 128)], sem)
                pltpu.async_copy(
                    embed_hbm.at[idx_v.at[c, pl.ds(128, L - 128)]],
                    rows_v.at[pl.ds(c * L + 128, L - 128)], sem)

        def drain(rows_v, sem):
            pltpu.make_async_copy(
                embed_hbm.at[pl.ds(0, _IDX_PER_CHUNK)], rows_v, sem).wait()

        def compute_store(rows_v, g):
            for c in range(_C):
                zero = jnp.zeros((16,), jnp.float32)
                accs = (zero,) * 8

                def row_body(j, a, c=c, rows_v=rows_v):
                    base = c * L + j * 4
                    lo = [a[u] + rows_v[base + u, pl.ds(0, 16)]
                          for u in range(4)]
                    hi = [a[4 + u] + rows_v[base + u, pl.ds(16, 16)]
                          for u in range(4)]
                    return tuple(lo + hi)

                accs = lax.fori_loop(0, L // 4, row_body, accs)
                sum_v[c, pl.ds(0, 16)] = (accs[0] + accs[1]) + (accs[2] + accs[3])
                sum_v[c, pl.ds(16, 16)] = (accs[4] + accs[5]) + (accs[6] + accs[7])
            pltpu.sync_copy(sum_v, out_hbm.at[pl.ds(worker_row0 + g * _C, _C)])

        fire(idx0, rows0, sem0, 0)

        def body(i, carry):
            g0 = 2 * i
            g1 = g0 + 1
            fire(idx1, rows1, sem1, g1)
            drain(rows0, sem0)
            compute_store(rows0, g0)

            @pl.when(g0 + 2 < _NCHUNK)
            def _():
                fire(idx0, rows0, sem0, g0 + 2)

            drain(rows1, sem1)
            compute_store(rows1, g1)
            return carry

        lax.fori_loop(0, _NCHUNK // 2, body, 0)

    return k(x, embed_lin)


def _tc_mlp(xT, summed, W1T, b1, W2T, b2):
    blk = 2048
    grid = (B // blk,)

    def body(xt_ref, s_ref, w1t_ref, b1_ref, w2t_ref, b2_ref, o_ref):
        cnt = jnp.sum((xt_ref[...] != 0).astype(jnp.float32), axis=0,
                      keepdims=True)
        avgT = s_ref[...].T / jnp.maximum(cnt, 1e-6)
        h = jnp.dot(w1t_ref[...], avgT, preferred_element_type=jnp.float32)
        h = jnp.maximum(h + b1_ref[...], 0.0)
        o = jnp.dot(w2t_ref[...], h, preferred_element_type=jnp.float32)
        o_ref[...] = o + b2_ref[...]

    return pl.pallas_call(
        body,
        grid=grid,
        in_specs=[
            pl.BlockSpec((L, blk), lambda i: (0, i)),
            pl.BlockSpec((blk, EMB), lambda i: (i, 0)),
            pl.BlockSpec((HID, EMB), lambda i: (0, 0)),
            pl.BlockSpec((HID, 1), lambda i: (0, 0)),
            pl.BlockSpec((NCLS, HID), lambda i: (0, 0)),
            pl.BlockSpec((NCLS, 1), lambda i: (0, 0)),
        ],
        out_specs=pl.BlockSpec((NCLS, blk), lambda i: (0, i)),
        out_shape=jax.ShapeDtypeStruct((NCLS, B), jnp.float32),
    )(xT, summed, W1T, b1.reshape(HID, 1), W2T, b2.reshape(NCLS, 1))


def kernel(x, embed, W1, b1, W2, b2):
    x = x.astype(jnp.int32)
    embed_lin = _tc_reformat(embed.T).reshape(_VPAD, EMB)
    fx = _tc_xform_idx(x)
    summed = _sc_segment_sum(fx, embed_lin)
    outT = _tc_mlp(x.T, summed, W1.T, b1, W2.T, b2)
    return outT.T


# speedup vs baseline: 1.9887x; 1.0974x over previous
import functools

import jax
import jax.numpy as jnp
from jax import lax
from jax.experimental import pallas as pl
from jax.experimental.pallas import tpu as pltpu
from jax.experimental.pallas import tpu_sc as plsc

VOCAB = 1000000
EMB = 32
HID = 128
NCLS = 100
B = 16384
L = 200

_NC = 2
_NS = 16
_NW = _NC * _NS
_BPW = B // _NW
_C = 8
_NCHUNK = _BPW // _C
_IDX_PER_CHUNK = _C * L

_RSUP = 512
_RK = 32
_RGRID = -(-VOCAB // _RSUP)
_VPAD = _RGRID * _RSUP


def _tc_reformat(embedT):
    def body(in_ref, o_ref):
        for k in range(_RK):
            t = jnp.concatenate(
                [in_ref[:, _RSUP * k + 128 * j:_RSUP * k + 128 * (j + 1)]
                 for j in range(4)], axis=0)
            o_ref[128 * k:128 * (k + 1), :] = t.T

    return pl.pallas_call(
        body,
        grid=(-(-_RGRID // _RK),),
        in_specs=[pl.BlockSpec((EMB, _RSUP * _RK), lambda i: (0, i))],
        out_specs=pl.BlockSpec((128 * _RK, 128), lambda i: (i, 0)),
        out_shape=jax.ShapeDtypeStruct((_VPAD // 4, 128), jnp.float32),
    )(embedT)


def _tc_xform_idx(x):
    blk = 2048
    grid = (B // blk,)

    def body(x_ref, o_ref):
        v = x_ref[...]
        o_ref[...] = (v & ~511) | ((v << 2) & 511) | ((v >> 7) & 3)

    return pl.pallas_call(
        body,
        grid=grid,
        in_specs=[pl.BlockSpec((blk, L), lambda i: (i, 0))],
        out_specs=pl.BlockSpec((blk, L), lambda i: (i, 0)),
        out_shape=jax.ShapeDtypeStruct((B, L), jnp.int32),
    )(x)


def _sc_segment_sum(x, embed_lin):
    mesh = plsc.VectorSubcoreMesh(core_axis_name="c", subcore_axis_name="s")

    @functools.partial(
        pl.kernel,
        out_type=jax.ShapeDtypeStruct((B, EMB), jnp.float32),
        mesh=mesh,
        compiler_params=pltpu.CompilerParams(use_tc_tiling_on_sc=False),
        scratch_types=[
            pltpu.VMEM((_C, L), jnp.int32),
            pltpu.VMEM((_C, L), jnp.int32),
            pltpu.VMEM((_IDX_PER_CHUNK, EMB), jnp.float32),
            pltpu.VMEM((_IDX_PER_CHUNK, EMB), jnp.float32),
            pltpu.VMEM((_C, EMB), jnp.float32),
            pltpu.SemaphoreType.DMA,
            pltpu.SemaphoreType.DMA,
        ],
    )
    def k(x_hbm, embed_hbm, out_hbm, idx0, idx1, rows0, rows1, sum_v,
          sem0, sem1):
        wid = lax.axis_index("s") * _NC + lax.axis_index("c")
        worker_row0 = wid * _BPW

        def fire(idx_v, rows_v, sem, g):
            pltpu.sync_copy(x_hbm.at[pl.ds(worker_row0 + g * _C, _C)], idx_v)
            for c in range(_C):
                pltpu.async_copy(
                    embed_hbm.at[idx_v.at[c, pl.ds(0, 128)]],
                    rows_v.at[pl.ds(c * L, 128)], sem)
                pltpu.async_copy(
                    embed_hbm.at[idx_v.at[c, pl.ds(128, L - 128)]],
                    rows_v.at[pl.ds(c * L + 128, L - 128)], sem)

        def drain(rows_v, sem):
            pltpu.make_async_copy(
                embed_hbm.at[pl.ds(0, _IDX_PER_CHUNK)], rows_v, sem).wait()

        def compute_store(rows_v, g):
            for c in range(_C):
                zero = jnp.zeros((16,), jnp.float32)
                accs = (zero,) * 8

                def row_body(j, a, c=c, rows_v=rows_v):
                    base = c * L + j * 4
                    lo = [a[u] + rows_v[base + u, pl.ds(0, 16)]
                          for u in range(4)]
                    hi = [a[4 + u] + rows_v[base + u, pl.ds(16, 16)]
                          for u in range(4)]
                    return tuple(lo + hi)

                accs = lax.fori_loop(0, L // 4, row_body, accs)
                sum_v[c, pl.ds(0, 16)] = (accs[0] + accs[1]) + (accs[2] + accs[3])
                sum_v[c, pl.ds(16, 16)] = (accs[4] + accs[5]) + (accs[6] + accs[7])
            pltpu.sync_copy(sum_v, out_hbm.at[pl.ds(worker_row0 + g * _C, _C)])

        fire(idx0, rows0, sem0, 0)

        def body(i, carry):
            g0 = 2 * i
            g1 = g0 + 1
            fire(idx1, rows1, sem1, g1)
            drain(rows0, sem0)
            compute_store(rows0, g0)

            @pl.when(g0 + 2 < _NCHUNK)
            def _():
                fire(idx0, rows0, sem0, g0 + 2)

            drain(rows1, sem1)
            compute_store(rows1, g1)
            return carry

        lax.fori_loop(0, _NCHUNK // 2, body, 0)

    return k(x, embed_lin)


def _tc_mlp(xT, summed, W1T, b1, W2T, b2):
    blk = 2048
    grid = (B // blk,)

    def body(xt_ref, s_ref, w1t_ref, b1_ref, w2t_ref, b2_ref, o_ref):
        cnt = jnp.sum((xt_ref[...] != 0).astype(jnp.float32), axis=0,
                      keepdims=True)
        avgT = s_ref[...].T / jnp.maximum(cnt, 1e-6)
        h = jnp.dot(w1t_ref[...], avgT, preferred_element_type=jnp.float32)
        h = jnp.maximum(h + b1_ref[...], 0.0)
        o = jnp.dot(w2t_ref[...], h, preferred_element_type=jnp.float32)
        o_ref[...] = o + b2_ref[...]

    return pl.pallas_call(
        body,
        grid=grid,
        in_specs=[
            pl.BlockSpec((L, blk), lambda i: (0, i)),
            pl.BlockSpec((blk, EMB), lambda i: (i, 0)),
            pl.BlockSpec((HID, EMB), lambda i: (0, 0)),
            pl.BlockSpec((HID, 1), lambda i: (0, 0)),
            pl.BlockSpec((NCLS, HID), lambda i: (0, 0)),
            pl.BlockSpec((NCLS, 1), lambda i: (0, 0)),
        ],
        out_specs=pl.BlockSpec((NCLS, blk), lambda i: (0, i)),
        out_shape=jax.ShapeDtypeStruct((NCLS, B), jnp.float32),
    )(xT, summed, W1T, b1.reshape(HID, 1), W2T, b2.reshape(NCLS, 1))


def kernel(x, embed, W1, b1, W2, b2):
    x = x.astype(jnp.int32)
    embed_lin = _tc_reformat(embed.T).reshape(_VPAD, EMB)
    fx = _tc_xform_idx(x)
    summed = _sc_segment_sum(fx, embed_lin)
    outT = _tc_mlp(x.T, summed, W1.T, b1, W2.T, b2)
    return outT.T
